# baseline (speedup 1.0000x reference)
import functools

import jax
import jax.numpy as jnp
from jax import lax
from jax.experimental import pallas as pl
from jax.experimental.pallas import tpu as pltpu
from jax.experimental.pallas import tpu_sc as plsc

KNN = 16
UPF = 4
SCALE = 256.0
F32 = jnp.float32
BF16 = jnp.bfloat16


def _mm(a, b):
    return lax.dot_general(a.astype(BF16), b.astype(BF16),
                           (((1,), (0,)), ((), ())),
                           preferred_element_type=F32)


def _gather_mm(oh, fc):
    return lax.dot_general(oh, fc, (((1,), (0,)), ((), ())),
                           precision=lax.Precision.HIGHEST,
                           preferred_element_type=F32)


def _dist2(xr, xaT):
    dot = lax.dot_general(xr.astype(BF16), xaT.astype(BF16),
                          (((1,), (0,)), ((), ())),
                          preferred_element_type=F32)
    sqr = jnp.sum(xr * xr, axis=1, keepdims=True)
    sqa = jnp.sum(xaT * xaT, axis=0, keepdims=True)
    return sqr + sqa - 2.0 * dot


def _extract_min(work, iota, m_cols):
    m = jnp.min(work, axis=1, keepdims=True)
    sel = jnp.min(jnp.where(work == m, iota, m_cols), axis=1, keepdims=True)
    ohb = iota == sel
    work = jnp.where(ohb, jnp.array(jnp.inf, F32), work)
    return ohb.astype(F32), sel, work


def _rnf_from_nb(nb, ctr):
    rel = nb - ctr
    dist = jnp.sqrt(jnp.sum(rel * rel, axis=1, keepdims=True))
    return jnp.concatenate([ctr, nb, rel, dist], axis=1)


def _lfa_h(nbf, rnf, wnb, bnb, wmx, bmx):
    nf = jnp.maximum(_mm(rnf, wnb) + bnb, 0.0)
    hin = jnp.concatenate([nbf, nf], axis=1)
    return jnp.maximum(_mm(hin, wmx) + bmx, 0.0)


def _knn_body(rows_ref, allT_ref, idx_ref):
    xr = rows_ref[0]
    xaT = allT_ref[0]
    m_cols = xaT.shape[1]
    d = _dist2(xr, xaT)
    iota = lax.broadcasted_iota(jnp.int32, d.shape, 1)
    sels = []
    for _ in range(KNN):
        _, sel, d = _extract_min(d, iota, m_cols)
        sels.append(sel)
    idx_ref[0] = jnp.concatenate(sels, axis=1) + pl.program_id(0) * m_cols


def _sc_gather(table, idx):
    R, D = table.shape
    btot = idx.shape[0]
    info = plsc.get_sparse_core_info()
    nw = info.num_cores * info.num_subcores
    b_per_w = btot // nw
    chunk = min(b_per_w, 1024)
    nchunks = b_per_w // chunk
    mesh = plsc.VectorSubcoreMesh(core_axis_name="c", subcore_axis_name="s")

    @functools.partial(
        pl.kernel, mesh=mesh,
        out_type=jax.ShapeDtypeStruct((btot, D), F32),
        compiler_params=pltpu.CompilerParams(use_tc_tiling_on_sc=False),
        scratch_types=[
            pltpu.VMEM((chunk,), jnp.int32),
            pltpu.VMEM((chunk, D), F32),
            pltpu.SemaphoreType.DMA,
        ],
    )
    def k(table_hbm, idx_hbm, out_hbm, idx_v, rows_v, sem):
        wid = lax.axis_index("s") * info.num_cores + lax.axis_index("c")
        base = wid * b_per_w

        def body(c, carry):
            off = base + c * chunk
            pltpu.sync_copy(idx_hbm.at[pl.ds(off, chunk)], idx_v)
            pltpu.async_copy(table_hbm.at[idx_v], rows_v, sem).wait()
            pltpu.sync_copy(rows_v, out_hbm.at[pl.ds(off, chunk)])
            return carry

        if nchunks == 1:
            body(0, 0)
        else:
            lax.fori_loop(0, nchunks, body, 0)

    return k(table, idx)


def _lfa2_body(ctr_ref, g_ref, wnb_ref, bnb_ref, wmx_ref, bmx_ref, out_ref):
    ctr = ctr_ref[...]
    cin = wmx_ref.shape[0] - 16
    acc = None
    for k in range(KNN):
        nb = g_ref[k]
        rnf = _rnf_from_nb(nb[:, :3], ctr)
        h = _lfa_h(nb[:, 3:3 + cin], rnf,
                   wnb_ref[...], bnb_ref[...], wmx_ref[...], bmx_ref[...])
        acc = h if acc is None else jnp.maximum(acc, h)
    out_ref[...] = acc


def _dec2_body(ctr_ref, g_ref, wnb_ref, bnb_ref, wmx_ref, bmx_ref,
               wpred_ref, bpred_ref, f_out_ref, coord_ref):
    ctr = ctr_ref[...]
    cin = wmx_ref.shape[0] - 16
    acc = None
    for k in range(KNN):
        nb = g_ref[k]
        rnf = _rnf_from_nb(nb[:, :3], ctr)
        h = _lfa_h(nb[:, 3:3 + cin], rnf,
                   wnb_ref[...], bnb_ref[...], wmx_ref[...], bmx_ref[...])
        acc = h if acc is None else jnp.maximum(acc, h)
    f_out_ref[...] = acc
    wpred, bpred = wpred_ref[...], bpred_ref[...]
    coords = []
    for c in range(UPF):
        off = _mm(acc[:, 32 * c:32 * (c + 1)], wpred) + bpred
        coords.append(ctr + off)
    coord_ref[...] = jnp.concatenate(coords, axis=1)


def _fin_body(cur_ref, curT_ref, feat_ref, w0nb, b0nb, w0mx, b0mx,
              w1nb, b1nb, w1mx, b1mx, ow_ref, ob_ref, out_ref):
    x = cur_ref[0]
    xT = curT_ref[0]
    f = feat_ref[0]
    m_cols = x.shape[0]
    d = _dist2(x, xT)
    iota = lax.broadcasted_iota(jnp.int32, d.shape, 1)
    fc = jnp.concatenate([x, f], axis=1)
    sels, rnfs = [], []
    acc = None
    for _ in range(KNN):
        oh, sel, d = _extract_min(d, iota, m_cols)
        sels.append(sel)
        g = _gather_mm(oh, fc)
        rnf = _rnf_from_nb(g[:, :3], x)
        rnfs.append(rnf)
        h = _lfa_h(g[:, 3:], rnf, w0nb[...], b0nb[...], w0mx[...], b0mx[...])
        acc = h if acc is None else jnp.maximum(acc, h)
    feat_a = acc
    acc = None
    for t in range(KNN):
        oh = (iota == sels[t]).astype(F32)
        g = _gather_mm(oh, feat_a)
        h = _lfa_h(g, rnfs[t], w1nb[...], b1nb[...], w1mx[...], b1mx[...])
        acc = h if acc is None else jnp.maximum(acc, h)
    y = (_mm(acc, ow_ref[...]) + ob_ref[...]) * SCALE
    yq = y + (jnp.round(y) - y)
    out_ref[0] = yq / SCALE


def _knn_call(cur, curT, bm):
    B, M, _ = cur.shape
    return pl.pallas_call(
        _knn_body,
        grid=(B, M // bm),
        compiler_params=pltpu.CompilerParams(
            dimension_semantics=("parallel", "parallel")),
        in_specs=[
            pl.BlockSpec((1, bm, 3), lambda b, i: (b, i, 0)),
            pl.BlockSpec((1, 3, M), lambda b, i: (b, 0, 0)),
        ],
        out_specs=pl.BlockSpec((1, bm, KNN), lambda b, i: (b, i, 0)),
        out_shape=jax.ShapeDtypeStruct((B, M, KNN), jnp.int32),
    )(cur, curT)


def _lfa2_call(cur_flat, g, p, bm):
    btot = cur_flat.shape[0]
    cout = p['wmx'].shape[-1]
    dcols = g.shape[-1]
    g = g.reshape(KNN, btot, dcols)
    return pl.pallas_call(
        _lfa2_body,
        grid=(btot // bm,),
        compiler_params=pltpu.CompilerParams(
            dimension_semantics=("parallel",)),
        in_specs=[
            pl.BlockSpec((bm, 3), lambda j: (j, 0)),
            pl.BlockSpec((KNN, bm, dcols), lambda j: (0, j, 0)),
            pl.BlockSpec((10, 16), lambda j: (0, 0)),
            pl.BlockSpec((1, 16), lambda j: (0, 0)),
            pl.BlockSpec(p['wmx'].shape, lambda j: (0, 0)),
            pl.BlockSpec((1, cout), lambda j: (0, 0)),
        ],
        out_specs=pl.BlockSpec((bm, cout), lambda j: (j, 0)),
        out_shape=jax.ShapeDtypeStruct((btot, cout), F32),
    )(cur_flat, g, p['wnb'], p['bnb'].reshape(1, -1),
      p['wmx'], p['bmx'].reshape(1, -1))


def _dec2_call(cur_flat, g, p, bm):
    btot = cur_flat.shape[0]
    cout = p['wmx'].shape[-1]
    dcols = g.shape[-1]
    g = g.reshape(KNN, btot, dcols)
    return pl.pallas_call(
        _dec2_body,
        grid=(btot // bm,),
        compiler_params=pltpu.CompilerParams(
            dimension_semantics=("parallel",)),
        in_specs=[
            pl.BlockSpec((bm, 3), lambda j: (j, 0)),
            pl.BlockSpec((KNN, bm, dcols), lambda j: (0, j, 0)),
            pl.BlockSpec((10, 16), lambda j: (0, 0)),
            pl.BlockSpec((1, 16), lambda j: (0, 0)),
            pl.BlockSpec(p['wmx'].shape, lambda j: (0, 0)),
            pl.BlockSpec((1, cout), lambda j: (0, 0)),
            pl.BlockSpec(p['wpred'].shape, lambda j: (0, 0)),
            pl.BlockSpec((1, 3), lambda j: (0, 0)),
        ],
        out_specs=[
            pl.BlockSpec((bm, cout), lambda j: (j, 0)),
            pl.BlockSpec((bm, 3 * UPF), lambda j: (j, 0)),
        ],
        out_shape=[
            jax.ShapeDtypeStruct((btot, cout), F32),
            jax.ShapeDtypeStruct((btot, 3 * UPF), F32),
        ],
    )(cur_flat, g, p['wnb'], p['bnb'].reshape(1, -1),
      p['wmx'], p['bmx'].reshape(1, -1),
      p['wpred'], p['bpred'].reshape(1, -1))


def _pad16(x):
    pad = (-x.shape[-1]) % 16
    if pad:
        x = jnp.pad(x, ((0, 0), (0, pad)))
    return x


def _fin_call(cur, curT, feat, p0, p1, ow, ob):
    B, M, _ = cur.shape
    cin = feat.shape[-1]
    comp = ow.shape[-1]
    args = (cur, curT, feat,
            p0['wnb'], p0['bnb'].reshape(1, -1), p0['wmx'], p0['bmx'].reshape(1, -1),
            p1['wnb'], p1['bnb'].reshape(1, -1), p1['wmx'], p1['bmx'].reshape(1, -1),
            ow, ob.reshape(1, -1))
    return pl.pallas_call(
        _fin_body,
        grid=(B,),
        in_specs=[
            pl.BlockSpec((1, M, 3), lambda b: (b, 0, 0)),
            pl.BlockSpec((1, 3, M), lambda b: (b, 0, 0)),
            pl.BlockSpec((1, M, cin), lambda b: (b, 0, 0)),
        ] + [pl.BlockSpec(a.shape, lambda b: (0, 0)) for a in args[3:]],
        out_specs=pl.BlockSpec((1, M, comp), lambda b: (b, 0, 0)),
        out_shape=jax.ShapeDtypeStruct((B, M, comp), F32),
    )(*args)


def kernel(xyz, params):
    B, N, _ = xyz.shape
    cur, feat = xyz, xyz

    for i in range(3):
        M = cur.shape[1]
        curT = jnp.swapaxes(cur, 1, 2)
        idx = _knn_call(cur, curT, min(M, 256))
        idxf = idx.reshape(B * M, KNN).T.reshape(-1)
        cur_flat = cur.reshape(B * M, 3)
        bm = min(B * M, 256)
        for half in ('a', 'b'):
            table = _pad16(jnp.concatenate(
                [cur_flat, feat.reshape(B * M, -1)], axis=-1))
            g = _sc_gather(table, idxf)
            feat = _lfa2_call(cur_flat, g, params['enc%d%s' % (i, half)], bm)
        feat = feat.reshape(B, M, -1)
        cur, feat = cur[:, ::UPF], feat[:, ::UPF]

    curT = jnp.swapaxes(cur, 1, 2)
    fea = _fin_call(cur, curT, feat, params['fin0'], params['fin1'],
                    params['out_w'], params['out_b'])

    for i in range(3):
        M = cur.shape[1]
        curT = jnp.swapaxes(cur, 1, 2)
        idx = _knn_call(cur, curT, min(M, 256))
        idxf = idx.reshape(B * M, KNN).T.reshape(-1)
        cur_flat = cur.reshape(B * M, 3)
        table = _pad16(jnp.concatenate(
            [cur_flat, fea.reshape(B * M, -1)], axis=-1))
        g = _sc_gather(table, idxf)
        f, coord12 = _dec2_call(cur_flat, g, params['dec%d' % i],
                                min(B * M, 256))
        fea = f.reshape(B, M * UPF, 32)
        cur = coord12.reshape(B, M * UPF, 3)
    return fea


# speedup vs baseline: 1.1563x; 1.1563x over previous
import functools

import jax
import jax.numpy as jnp
from jax import lax
from jax.experimental import pallas as pl
from jax.experimental.pallas import tpu as pltpu
from jax.experimental.pallas import tpu_sc as plsc

KNN = 16
UPF = 4
SCALE = 256.0
F32 = jnp.float32
BF16 = jnp.bfloat16


def _mm(a, b):
    return lax.dot_general(a.astype(BF16), b.astype(BF16),
                           (((1,), (0,)), ((), ())),
                           preferred_element_type=F32)


def _gather_mm(oh, fc):
    return lax.dot_general(oh, fc, (((1,), (0,)), ((), ())),
                           precision=lax.Precision.HIGHEST,
                           preferred_element_type=F32)


def _dist2(xr, xaT):
    dot = lax.dot_general(xr.astype(BF16), xaT.astype(BF16),
                          (((1,), (0,)), ((), ())),
                          preferred_element_type=F32)
    sqr = jnp.sum(xr * xr, axis=1, keepdims=True)
    sqa = jnp.sum(xaT * xaT, axis=0, keepdims=True)
    return sqr + sqa - 2.0 * dot


def _extract_min(work, iota, m_cols):
    m = jnp.min(work, axis=1, keepdims=True)
    sel = jnp.min(jnp.where(work == m, iota, m_cols), axis=1, keepdims=True)
    ohb = iota == sel
    work = jnp.where(ohb, jnp.array(jnp.inf, F32), work)
    return ohb.astype(F32), sel, work


def _rnf_from_nb(nb, ctr):
    rel = nb - ctr
    dist = jnp.sqrt(jnp.sum(rel * rel, axis=1, keepdims=True))
    return jnp.concatenate([ctr, nb, rel, dist], axis=1)


def _lfa_h(nbf, rnf, wnb, bnb, wmx, bmx):
    nf = jnp.maximum(_mm(rnf, wnb) + bnb, 0.0)
    hin = jnp.concatenate([nbf, nf], axis=1)
    return jnp.maximum(_mm(hin, wmx) + bmx, 0.0)


def _knn_body(rows_ref, allT_ref, idx_ref):
    xr = rows_ref[0]
    xaT = allT_ref[0]
    m_cols = xaT.shape[1]
    d = _dist2(xr, xaT)
    iota = lax.broadcasted_iota(jnp.int32, d.shape, 1)
    sels = []
    for _ in range(KNN):
        _, sel, d = _extract_min(d, iota, m_cols)
        sels.append(sel)
    idx_ref[0] = jnp.concatenate(sels, axis=1) + pl.program_id(0) * m_cols


def _sc_gather(table, idx):
    R, D = table.shape
    btot = idx.shape[0]
    info = plsc.get_sparse_core_info()
    nw = info.num_cores * info.num_subcores
    b_per_w = btot // nw
    chunk = min(b_per_w, 1024)
    nchunks = b_per_w // chunk
    mesh = plsc.VectorSubcoreMesh(core_axis_name="c", subcore_axis_name="s")

    @functools.partial(
        pl.kernel, mesh=mesh,
        out_type=jax.ShapeDtypeStruct((btot, D), F32),
        compiler_params=pltpu.CompilerParams(use_tc_tiling_on_sc=False),
        scratch_types=[
            pltpu.VMEM((chunk,), jnp.int32),
            pltpu.VMEM((chunk, D), F32),
            pltpu.SemaphoreType.DMA,
        ],
    )
    def k(table_hbm, idx_hbm, out_hbm, idx_v, rows_v, sem):
        wid = lax.axis_index("s") * info.num_cores + lax.axis_index("c")
        base = wid * b_per_w

        def body(c, carry):
            off = base + c * chunk
            pltpu.sync_copy(idx_hbm.at[pl.ds(off, chunk)], idx_v)
            pltpu.async_copy(table_hbm.at[idx_v], rows_v, sem).wait()
            pltpu.sync_copy(rows_v, out_hbm.at[pl.ds(off, chunk)])
            return carry

        if nchunks == 1:
            body(0, 0)
        else:
            lax.fori_loop(0, nchunks, body, 0)

    return k(table, idx)


def _lfa2_body(ctr_ref, g_ref, wnb_ref, bnb_ref, wmx_ref, bmx_ref, out_ref):
    ctr = ctr_ref[...]
    g = g_ref[...]
    bm = ctr.shape[0]
    ctrk = jnp.broadcast_to(ctr[:, None, :], (bm, KNN, 3)).reshape(bm * KNN, 3)
    rnf = _rnf_from_nb(g[:, :3], ctrk)
    h = _lfa_h(g[:, 3:3 + (wmx_ref.shape[0] - 16)], rnf,
               wnb_ref[...], bnb_ref[...], wmx_ref[...], bmx_ref[...])
    out_ref[...] = jnp.max(h.reshape(bm, KNN, h.shape[1]), axis=1)


def _dec2_body(ctr_ref, g_ref, wnb_ref, bnb_ref, wmx_ref, bmx_ref,
               wpred_ref, bpred_ref, f_out_ref, coord_ref):
    ctr = ctr_ref[...]
    g = g_ref[...]
    bm = ctr.shape[0]
    ctrk = jnp.broadcast_to(ctr[:, None, :], (bm, KNN, 3)).reshape(bm * KNN, 3)
    rnf = _rnf_from_nb(g[:, :3], ctrk)
    h = _lfa_h(g[:, 3:3 + (wmx_ref.shape[0] - 16)], rnf,
               wnb_ref[...], bnb_ref[...], wmx_ref[...], bmx_ref[...])
    acc = jnp.max(h.reshape(bm, KNN, h.shape[1]), axis=1)
    f_out_ref[...] = acc
    wpred, bpred = wpred_ref[...], bpred_ref[...]
    coords = []
    for c in range(UPF):
        off = _mm(acc[:, 32 * c:32 * (c + 1)], wpred) + bpred
        coords.append(ctr + off)
    coord_ref[...] = jnp.concatenate(coords, axis=1)


def _fin_body(cur_ref, curT_ref, feat_ref, w0nb, b0nb, w0mx, b0mx,
              w1nb, b1nb, w1mx, b1mx, ow_ref, ob_ref, out_ref):
    x = cur_ref[0]
    xT = curT_ref[0]
    f = feat_ref[0]
    m_cols = x.shape[0]
    d = _dist2(x, xT)
    iota = lax.broadcasted_iota(jnp.int32, d.shape, 1)
    fc = jnp.concatenate([x, f], axis=1)
    sels, rnfs = [], []
    acc = None
    for _ in range(KNN):
        oh, sel, d = _extract_min(d, iota, m_cols)
        sels.append(sel)
        g = _gather_mm(oh, fc)
        rnf = _rnf_from_nb(g[:, :3], x)
        rnfs.append(rnf)
        h = _lfa_h(g[:, 3:], rnf, w0nb[...], b0nb[...], w0mx[...], b0mx[...])
        acc = h if acc is None else jnp.maximum(acc, h)
    feat_a = acc
    acc = None
    for t in range(KNN):
        oh = (iota == sels[t]).astype(F32)
        g = _gather_mm(oh, feat_a)
        h = _lfa_h(g, rnfs[t], w1nb[...], b1nb[...], w1mx[...], b1mx[...])
        acc = h if acc is None else jnp.maximum(acc, h)
    y = (_mm(acc, ow_ref[...]) + ob_ref[...]) * SCALE
    yq = y + (jnp.round(y) - y)
    out_ref[0] = yq / SCALE


def _tail_body(cur_ref, curT_ref, feat_ref,
               e2a_wnb, e2a_bnb, e2a_wmx, e2a_bmx,
               e2b_wnb, e2b_bnb, e2b_wmx, e2b_bmx,
               f0_wnb, f0_bnb, f0_wmx, f0_bmx,
               f1_wnb, f1_bnb, f1_wmx, f1_bmx,
               ow_ref, ob_ref,
               d0_wnb, d0_bnb, d0_wmx, d0_bmx, d0_wp, d0_bp,
               d1_wnb, d1_bnb, d1_wmx, d1_bmx, d1_wp, d1_bp,
               fea_ref, coord_ref):
    x2 = cur_ref[0]
    x2T = curT_ref[0]
    f2 = feat_ref[0]
    m2 = x2.shape[0]
    m6 = m2 // UPF

    d = _dist2(x2, x2T)
    iota = lax.broadcasted_iota(jnp.int32, d.shape, 1)
    fc = jnp.concatenate([x2, f2], axis=1)
    ohs, rnfs = [], []
    acc = None
    for _ in range(KNN):
        oh, _, d = _extract_min(d, iota, m2)
        ohs.append(oh)
        g = _gather_mm(oh, fc)
        rnf = _rnf_from_nb(g[:, :3], x2)
        rnfs.append(rnf)
        h = _lfa_h(g[:, 3:], rnf, e2a_wnb[...], e2a_bnb[...],
                   e2a_wmx[...], e2a_bmx[...])
        acc = h if acc is None else jnp.maximum(acc, h)
    fa = acc
    acc = None
    for t in range(KNN):
        g = _gather_mm(ohs[t], fa)
        h = _lfa_h(g, rnfs[t], e2b_wnb[...], e2b_bnb[...],
                   e2b_wmx[...], e2b_bmx[...])
        acc = h if acc is None else jnp.maximum(acc, h)
    f2b = acc

    r64 = lax.broadcasted_iota(jnp.int32, (m6, m2), 0)
    c64 = lax.broadcasted_iota(jnp.int32, (m6, m2), 1)
    S = (c64 == UPF * r64).astype(F32)
    rT = lax.broadcasted_iota(jnp.int32, (m2, m6), 0)
    cT = lax.broadcasted_iota(jnp.int32, (m2, m6), 1)
    ST = (rT == UPF * cT).astype(F32)
    x6 = _gather_mm(S, x2)
    x6T = _gather_mm(x2T, ST)
    f6 = _gather_mm(S, f2b)

    d6 = _dist2(x6, x6T)
    iota6 = lax.broadcasted_iota(jnp.int32, d6.shape, 1)
    ohs6, rnfs6 = [], []
    acc = None
    for _ in range(KNN):
        oh, _, d6 = _extract_min(d6, iota6, m6)
        ohs6.append(oh)
        g = _gather_mm(oh, jnp.concatenate([x6, f6], axis=1))
        rnf = _rnf_from_nb(g[:, :3], x6)
        rnfs6.append(rnf)
        h = _lfa_h(g[:, 3:], rnf, f0_wnb[...], f0_bnb[...],
                   f0_wmx[...], f0_bmx[...])
        acc = h if acc is None else jnp.maximum(acc, h)
    feat_a = acc
    acc = None
    for t in range(KNN):
        g = _gather_mm(ohs6[t], feat_a)
        h = _lfa_h(g, rnfs6[t], f1_wnb[...], f1_bnb[...],
                   f1_wmx[...], f1_bmx[...])
        acc = h if acc is None else jnp.maximum(acc, h)
    y = (_mm(acc, ow_ref[...]) + ob_ref[...]) * SCALE
    yq = (y + (jnp.round(y) - y)) / SCALE

    acc = None
    for t in range(KNN):
        g = _gather_mm(ohs6[t], yq)
        h = _lfa_h(g, rnfs6[t], d0_wnb[...], d0_bnb[...],
                   d0_wmx[...], d0_bmx[...])
        acc = h if acc is None else jnp.maximum(acc, h)
    cur1 = None
    fea1 = None
    for c in range(UPF):
        rc = lax.broadcasted_iota(jnp.int32, (m2, m6), 0)
        cc = lax.broadcasted_iota(jnp.int32, (m2, m6), 1)
        Rc = (rc == UPF * cc + c).astype(F32)
        off = _mm(acc[:, 32 * c:32 * (c + 1)], d0_wp[...]) + d0_bp[...]
        contrib_x = _gather_mm(Rc, x6 + off)
        contrib_f = _gather_mm(Rc, acc[:, 32 * c:32 * (c + 1)])
        cur1 = contrib_x if cur1 is None else cur1 + contrib_x
        fea1 = contrib_f if fea1 is None else fea1 + contrib_f

    dot1 = lax.dot_general(cur1.astype(BF16), cur1.astype(BF16),
                           (((1,), (1,)), ((), ())),
                           preferred_element_type=F32)
    sq1 = jnp.sum(cur1 * cur1, axis=1, keepdims=True)
    sq1r = lax.dot_general(jnp.ones((1, 1), F32), sq1,
                           (((1,), (1,)), ((), ())),
                           precision=lax.Precision.HIGHEST,
                           preferred_element_type=F32)
    d1 = sq1 + sq1r - 2.0 * dot1
    fc1 = jnp.concatenate([cur1, fea1], axis=1)
    acc = None
    for _ in range(KNN):
        oh, _, d1 = _extract_min(d1, iota, m2)
        g = _gather_mm(oh, fc1)
        rnf = _rnf_from_nb(g[:, :3], cur1)
        h = _lfa_h(g[:, 3:], rnf, d1_wnb[...], d1_bnb[...],
                   d1_wmx[...], d1_bmx[...])
        acc = h if acc is None else jnp.maximum(acc, h)
    fea_ref[0] = acc
    coords = []
    for c in range(UPF):
        off = _mm(acc[:, 32 * c:32 * (c + 1)], d1_wp[...]) + d1_bp[...]
        coords.append(cur1 + off)
    coord_ref[0] = jnp.concatenate(coords, axis=1)


def _tail_call(cur, curT, feat, params):
    B = cur.shape[0]
    args = [cur, curT, feat]
    for name in ('enc2a', 'enc2b', 'fin0', 'fin1'):
        p = params[name]
        args += [p['wnb'], p['bnb'].reshape(1, -1),
                 p['wmx'], p['bmx'].reshape(1, -1)]
    args += [params['out_w'], params['out_b'].reshape(1, -1)]
    for name in ('dec0', 'dec1'):
        p = params[name]
        args += [p['wnb'], p['bnb'].reshape(1, -1),
                 p['wmx'], p['bmx'].reshape(1, -1),
                 p['wpred'], p['bpred'].reshape(1, -1)]
    in_specs = [
        pl.BlockSpec((1, 256, 3), lambda b: (b, 0, 0)),
        pl.BlockSpec((1, 3, 256), lambda b: (b, 0, 0)),
        pl.BlockSpec((1, 256, 64), lambda b: (b, 0, 0)),
    ] + [pl.BlockSpec(a.shape, lambda b: (0, 0)) for a in args[3:]]
    return pl.pallas_call(
        _tail_body,
        grid=(B,),
        in_specs=in_specs,
        out_specs=[
            pl.BlockSpec((1, 256, 128), lambda b: (b, 0, 0)),
            pl.BlockSpec((1, 256, 12), lambda b: (b, 0, 0)),
        ],
        out_shape=[
            jax.ShapeDtypeStruct((B, 256, 128), F32),
            jax.ShapeDtypeStruct((B, 256, 12), F32),
        ],
    )(*args)


def _knn_call(cur, curT, bm):
    B, M, _ = cur.shape
    return pl.pallas_call(
        _knn_body,
        grid=(B, M // bm),
        compiler_params=pltpu.CompilerParams(
            dimension_semantics=("parallel", "parallel")),
        in_specs=[
            pl.BlockSpec((1, bm, 3), lambda b, i: (b, i, 0)),
            pl.BlockSpec((1, 3, M), lambda b, i: (b, 0, 0)),
        ],
        out_specs=pl.BlockSpec((1, bm, KNN), lambda b, i: (b, i, 0)),
        out_shape=jax.ShapeDtypeStruct((B, M, KNN), jnp.int32),
    )(cur, curT)


def _lfa2_call(cur_flat, g, p, bm):
    btot = cur_flat.shape[0]
    cout = p['wmx'].shape[-1]
    dcols = g.shape[-1]
    return pl.pallas_call(
        _lfa2_body,
        grid=(btot // bm,),
        compiler_params=pltpu.CompilerParams(
            dimension_semantics=("parallel",)),
        in_specs=[
            pl.BlockSpec((bm, 3), lambda j: (j, 0)),
            pl.BlockSpec((bm * KNN, dcols), lambda j: (j, 0)),
            pl.BlockSpec((10, 16), lambda j: (0, 0)),
            pl.BlockSpec((1, 16), lambda j: (0, 0)),
            pl.BlockSpec(p['wmx'].shape, lambda j: (0, 0)),
            pl.BlockSpec((1, cout), lambda j: (0, 0)),
        ],
        out_specs=pl.BlockSpec((bm, cout), lambda j: (j, 0)),
        out_shape=jax.ShapeDtypeStruct((btot, cout), F32),
    )(cur_flat, g, p['wnb'], p['bnb'].reshape(1, -1),
      p['wmx'], p['bmx'].reshape(1, -1))


def _dec2_call(cur_flat, g, p, bm):
    btot = cur_flat.shape[0]
    cout = p['wmx'].shape[-1]
    dcols = g.shape[-1]
    return pl.pallas_call(
        _dec2_body,
        grid=(btot // bm,),
        compiler_params=pltpu.CompilerParams(
            dimension_semantics=("parallel",)),
        in_specs=[
            pl.BlockSpec((bm, 3), lambda j: (j, 0)),
            pl.BlockSpec((bm * KNN, dcols), lambda j: (j, 0)),
            pl.BlockSpec((10, 16), lambda j: (0, 0)),
            pl.BlockSpec((1, 16), lambda j: (0, 0)),
            pl.BlockSpec(p['wmx'].shape, lambda j: (0, 0)),
            pl.BlockSpec((1, cout), lambda j: (0, 0)),
            pl.BlockSpec(p['wpred'].shape, lambda j: (0, 0)),
            pl.BlockSpec((1, 3), lambda j: (0, 0)),
        ],
        out_specs=[
            pl.BlockSpec((bm, cout), lambda j: (j, 0)),
            pl.BlockSpec((bm, 3 * UPF), lambda j: (j, 0)),
        ],
        out_shape=[
            jax.ShapeDtypeStruct((btot, cout), F32),
            jax.ShapeDtypeStruct((btot, 3 * UPF), F32),
        ],
    )(cur_flat, g, p['wnb'], p['bnb'].reshape(1, -1),
      p['wmx'], p['bmx'].reshape(1, -1),
      p['wpred'], p['bpred'].reshape(1, -1))


def _pad16(x):
    pad = (-x.shape[-1]) % 16
    if pad:
        x = jnp.pad(x, ((0, 0), (0, pad)))
    return x


def _fin_call(cur, curT, feat, p0, p1, ow, ob):
    B, M, _ = cur.shape
    cin = feat.shape[-1]
    comp = ow.shape[-1]
    args = (cur, curT, feat,
            p0['wnb'], p0['bnb'].reshape(1, -1), p0['wmx'], p0['bmx'].reshape(1, -1),
            p1['wnb'], p1['bnb'].reshape(1, -1), p1['wmx'], p1['bmx'].reshape(1, -1),
            ow, ob.reshape(1, -1))
    return pl.pallas_call(
        _fin_body,
        grid=(B,),
        in_specs=[
            pl.BlockSpec((1, M, 3), lambda b: (b, 0, 0)),
            pl.BlockSpec((1, 3, M), lambda b: (b, 0, 0)),
            pl.BlockSpec((1, M, cin), lambda b: (b, 0, 0)),
        ] + [pl.BlockSpec(a.shape, lambda b: (0, 0)) for a in args[3:]],
        out_specs=pl.BlockSpec((1, M, comp), lambda b: (b, 0, 0)),
        out_shape=jax.ShapeDtypeStruct((B, M, comp), F32),
    )(*args)


def kernel(xyz, params):
    B, N, _ = xyz.shape
    cur, feat = xyz, xyz

    for i in range(2):
        M = cur.shape[1]
        curT = jnp.swapaxes(cur, 1, 2)
        idx = _knn_call(cur, curT, min(M, 256))
        idxf = idx.reshape(-1)
        cur_flat = cur.reshape(B * M, 3)
        bm = min(B * M, 256)
        for half in ('a', 'b'):
            table = _pad16(jnp.concatenate(
                [cur_flat, feat.reshape(B * M, -1)], axis=-1))
            g = _sc_gather(table, idxf)
            feat = _lfa2_call(cur_flat, g, params['enc%d%s' % (i, half)], bm)
        feat = feat.reshape(B, M, -1)
        cur, feat = cur[:, ::UPF], feat[:, ::UPF]

    curT = jnp.swapaxes(cur, 1, 2)
    fea2, coord12 = _tail_call(cur, curT, feat, params)
    fea = fea2.reshape(B, 1024, 32)
    cur = coord12.reshape(B, 1024, 3)

    M = cur.shape[1]
    curT = jnp.swapaxes(cur, 1, 2)
    idx = _knn_call(cur, curT, 256)
    idxf = idx.reshape(-1)
    cur_flat = cur.reshape(B * M, 3)
    table = _pad16(jnp.concatenate(
        [cur_flat, fea.reshape(B * M, -1)], axis=-1))
    g = _sc_gather(table, idxf)
    f, _ = _dec2_call(cur_flat, g, params['dec2'], 256)
    return f.reshape(B, M * UPF, 32)


# speedup vs baseline: 1.2027x; 1.0401x over previous
import functools

import jax
import jax.numpy as jnp
from jax import lax
from jax.experimental import pallas as pl
from jax.experimental.pallas import tpu as pltpu
from jax.experimental.pallas import tpu_sc as plsc

KNN = 16
UPF = 4
SCALE = 256.0
F32 = jnp.float32
BF16 = jnp.bfloat16


def _mm(a, b):
    return lax.dot_general(a.astype(BF16), b.astype(BF16),
                           (((1,), (0,)), ((), ())),
                           preferred_element_type=F32)


def _gather_mm(oh, fc):
    return lax.dot_general(oh, fc, (((1,), (0,)), ((), ())),
                           precision=lax.Precision.HIGHEST,
                           preferred_element_type=F32)


def _dist2(xr, xaT):
    dot = _mm(xr, xaT)
    sqr = jnp.sum(xr * xr, axis=1, keepdims=True)
    sqa = jnp.sum(xaT * xaT, axis=0, keepdims=True)
    return sqr + sqa - 2.0 * dot


def _extract_min(work, iota, m_cols):
    m = jnp.min(work, axis=1, keepdims=True)
    sel = jnp.min(jnp.where(work == m, iota, m_cols), axis=1, keepdims=True)
    ohb = iota == sel
    work = jnp.where(ohb, jnp.array(jnp.inf, F32), work)
    return ohb.astype(F32), sel, work


def _rnf_from_nb(nb, ctr):
    rel = nb - ctr
    dist = jnp.sqrt(jnp.sum(rel * rel, axis=1, keepdims=True))
    return jnp.concatenate([ctr, nb, rel, dist], axis=1)


def _lfa_h(nbf, rnf, wnb, bnb, wmx, bmx):
    nf = jnp.maximum(_mm(rnf, wnb) + bnb, 0.0)
    hin = jnp.concatenate([nbf, nf], axis=1)
    return jnp.maximum(_mm(hin, wmx) + bmx, 0.0)


def _knn_body(rows_ref, allT_ref, idx_ref):
    xr = rows_ref[0]
    xaT = allT_ref[0]
    m_cols = xaT.shape[1]
    base = pl.program_id(0) * m_cols
    d = _dist2(xr, xaT)
    iota = lax.broadcasted_iota(jnp.int32, d.shape, 1)
    sels = []
    for _ in range(KNN):
        _, sel, d = _extract_min(d, iota, m_cols)
        sels.append(sel)
    idx_ref[0] = jnp.concatenate(sels, axis=1) + base


def _sc_gather(table, idx):
    R, D = table.shape
    btot = idx.shape[0]
    info = plsc.get_sparse_core_info()
    nw = info.num_cores * info.num_subcores
    b_per_w = btot // nw
    chunk = min(b_per_w, 1024)
    nchunks = b_per_w // chunk
    mesh = plsc.VectorSubcoreMesh(core_axis_name="c", subcore_axis_name="s")

    @functools.partial(
        pl.kernel, mesh=mesh,
        out_type=jax.ShapeDtypeStruct((btot, D), F32),
        compiler_params=pltpu.CompilerParams(use_tc_tiling_on_sc=False),
        scratch_types=[
            pltpu.VMEM((chunk,), jnp.int32),
            pltpu.VMEM((chunk, D), F32),
            pltpu.SemaphoreType.DMA,
        ],
    )
    def k(table_hbm, idx_hbm, out_hbm, idx_v, rows_v, sem):
        wid = lax.axis_index("s") * info.num_cores + lax.axis_index("c")
        base = wid * b_per_w

        def body(c, carry):
            off = base + c * chunk
            pltpu.sync_copy(idx_hbm.at[pl.ds(off, chunk)], idx_v)
            pltpu.async_copy(table_hbm.at[idx_v], rows_v, sem).wait()
            pltpu.sync_copy(rows_v, out_hbm.at[pl.ds(off, chunk)])
            return carry

        if nchunks == 1:
            body(0, 0)
        else:
            lax.fori_loop(0, nchunks, body, 0)

    return k(table, idx)


def _lfa2_body(ctr_ref, g_ref, wnb_ref, bnb_ref, wmx_ref, bmx_ref, out_ref):
    ctr = ctr_ref[...]
    g = g_ref[...]
    bm = ctr.shape[0]
    ctrk = jnp.broadcast_to(ctr[:, None, :], (bm, KNN, 3)).reshape(bm * KNN, 3)
    rnf = _rnf_from_nb(g[:, :3], ctrk)
    h = _lfa_h(g[:, 3:3 + (wmx_ref.shape[0] - 16)], rnf,
               wnb_ref[...], bnb_ref[...], wmx_ref[...], bmx_ref[...])
    out_ref[...] = jnp.max(h.reshape(bm, KNN, h.shape[1]), axis=1)


def _dec2_body(ctr_ref, g_ref, wnb_ref, bnb_ref, wmx_ref, bmx_ref,
               wpred_ref, bpred_ref, f_out_ref, coord_ref):
    ctr = ctr_ref[...]
    g = g_ref[...]
    bm = ctr.shape[0]
    ctrk = jnp.broadcast_to(ctr[:, None, :], (bm, KNN, 3)).reshape(bm * KNN, 3)
    rnf = _rnf_from_nb(g[:, :3], ctrk)
    h = _lfa_h(g[:, 3:3 + (wmx_ref.shape[0] - 16)], rnf,
               wnb_ref[...], bnb_ref[...], wmx_ref[...], bmx_ref[...])
    acc = jnp.max(h.reshape(bm, KNN, h.shape[1]), axis=1)
    f_out_ref[...] = acc
    wpred, bpred = wpred_ref[...], bpred_ref[...]
    coords = []
    for c in range(UPF):
        off = _mm(acc[:, 32 * c:32 * (c + 1)], wpred) + bpred
        coords.append(ctr + off)
    coord_ref[...] = jnp.concatenate(coords, axis=1)


def _fin_body(cur_ref, curT_ref, feat_ref, w0nb, b0nb, w0mx, b0mx,
              w1nb, b1nb, w1mx, b1mx, ow_ref, ob_ref, out_ref):
    x = cur_ref[0]
    xT = curT_ref[0]
    f = feat_ref[0]
    m_cols = x.shape[0]
    d = _dist2(x, xT)
    iota = lax.broadcasted_iota(jnp.int32, d.shape, 1)
    fc = jnp.concatenate([x, f], axis=1)
    sels, rnfs = [], []
    acc = None
    for _ in range(KNN):
        oh, sel, d = _extract_min(d, iota, m_cols)
        sels.append(sel)
        g = _gather_mm(oh, fc)
        rnf = _rnf_from_nb(g[:, :3], x)
        rnfs.append(rnf)
        h = _lfa_h(g[:, 3:], rnf, w0nb[...], b0nb[...], w0mx[...], b0mx[...])
        acc = h if acc is None else jnp.maximum(acc, h)
    feat_a = acc
    acc = None
    for t in range(KNN):
        oh = (iota == sels[t]).astype(F32)
        g = _gather_mm(oh, feat_a)
        h = _lfa_h(g, rnfs[t], w1nb[...], b1nb[...], w1mx[...], b1mx[...])
        acc = h if acc is None else jnp.maximum(acc, h)
    y = (_mm(acc, ow_ref[...]) + ob_ref[...]) * SCALE
    yq = y + (jnp.round(y) - y)
    out_ref[0] = yq / SCALE


def _tail_body(cur_ref, curT_ref, feat_ref,
               e2a_wnb, e2a_bnb, e2a_wmx, e2a_bmx,
               e2b_wnb, e2b_bnb, e2b_wmx, e2b_bmx,
               f0_wnb, f0_bnb, f0_wmx, f0_bmx,
               f1_wnb, f1_bnb, f1_wmx, f1_bmx,
               ow_ref, ob_ref,
               d0_wnb, d0_bnb, d0_wmx, d0_bmx, d0_wp, d0_bp,
               d1_wnb, d1_bnb, d1_wmx, d1_bmx, d1_wp, d1_bp,
               fea_ref, coord_ref):
    x2 = cur_ref[0]
    x2T = curT_ref[0]
    f2 = feat_ref[0]
    m2 = x2.shape[0]
    m6 = m2 // UPF

    d = _dist2(x2, x2T)
    iota = lax.broadcasted_iota(jnp.int32, d.shape, 1)
    fc = jnp.concatenate([x2, f2], axis=1)
    ohs, rnfs = [], []
    acc = None
    for _ in range(KNN):
        oh, _, d = _extract_min(d, iota, m2)
        ohs.append(oh)
        g = _gather_mm(oh, fc)
        rnf = _rnf_from_nb(g[:, :3], x2)
        rnfs.append(rnf)
        h = _lfa_h(g[:, 3:], rnf, e2a_wnb[...], e2a_bnb[...],
                   e2a_wmx[...], e2a_bmx[...])
        acc = h if acc is None else jnp.maximum(acc, h)
    fa = acc
    acc = None
    for t in range(KNN):
        g = _gather_mm(ohs[t], fa)
        h = _lfa_h(g, rnfs[t], e2b_wnb[...], e2b_bnb[...],
                   e2b_wmx[...], e2b_bmx[...])
        acc = h if acc is None else jnp.maximum(acc, h)
    f2b = acc

    r64 = lax.broadcasted_iota(jnp.int32, (m6, m2), 0)
    c64 = lax.broadcasted_iota(jnp.int32, (m6, m2), 1)
    S = (c64 == UPF * r64).astype(F32)
    rT = lax.broadcasted_iota(jnp.int32, (m2, m6), 0)
    cT = lax.broadcasted_iota(jnp.int32, (m2, m6), 1)
    ST = (rT == UPF * cT).astype(F32)
    x6 = _gather_mm(S, x2)
    x6T = _gather_mm(x2T, ST)
    f6 = _gather_mm(S, f2b)

    d6 = _dist2(x6, x6T)
    iota6 = lax.broadcasted_iota(jnp.int32, d6.shape, 1)
    ohs6, rnfs6 = [], []
    acc = None
    for _ in range(KNN):
        oh, _, d6 = _extract_min(d6, iota6, m6)
        ohs6.append(oh)
        g = _gather_mm(oh, jnp.concatenate([x6, f6], axis=1))
        rnf = _rnf_from_nb(g[:, :3], x6)
        rnfs6.append(rnf)
        h = _lfa_h(g[:, 3:], rnf, f0_wnb[...], f0_bnb[...],
                   f0_wmx[...], f0_bmx[...])
        acc = h if acc is None else jnp.maximum(acc, h)
    feat_a = acc
    acc = None
    for t in range(KNN):
        g = _gather_mm(ohs6[t], feat_a)
        h = _lfa_h(g, rnfs6[t], f1_wnb[...], f1_bnb[...],
                   f1_wmx[...], f1_bmx[...])
        acc = h if acc is None else jnp.maximum(acc, h)
    y = (_mm(acc, ow_ref[...]) + ob_ref[...]) * SCALE
    yq = (y + (jnp.round(y) - y)) / SCALE

    acc = None
    for t in range(KNN):
        g = _gather_mm(ohs6[t], yq)
        h = _lfa_h(g, rnfs6[t], d0_wnb[...], d0_bnb[...],
                   d0_wmx[...], d0_bmx[...])
        acc = h if acc is None else jnp.maximum(acc, h)
    cur1 = None
    fea1 = None
    for c in range(UPF):
        rc = lax.broadcasted_iota(jnp.int32, (m2, m6), 0)
        cc = lax.broadcasted_iota(jnp.int32, (m2, m6), 1)
        Rc = (rc == UPF * cc + c).astype(F32)
        off = _mm(acc[:, 32 * c:32 * (c + 1)], d0_wp[...]) + d0_bp[...]
        contrib_x = _gather_mm(Rc, x6 + off)
        contrib_f = _gather_mm(Rc, acc[:, 32 * c:32 * (c + 1)])
        cur1 = contrib_x if cur1 is None else cur1 + contrib_x
        fea1 = contrib_f if fea1 is None else fea1 + contrib_f

    dot1 = lax.dot_general(cur1.astype(BF16), cur1.astype(BF16),
                           (((1,), (1,)), ((), ())),
                           preferred_element_type=F32)
    sq1 = jnp.sum(cur1 * cur1, axis=1, keepdims=True)
    sq1r = lax.dot_general(jnp.ones((1, 1), F32), sq1,
                           (((1,), (1,)), ((), ())),
                           precision=lax.Precision.HIGHEST,
                           preferred_element_type=F32)
    d1 = sq1 + sq1r - 2.0 * dot1
    fc1 = jnp.concatenate([cur1, fea1], axis=1)
    acc = None
    for _ in range(KNN):
        oh, _, d1 = _extract_min(d1, iota, m2)
        g = _gather_mm(oh, fc1)
        rnf = _rnf_from_nb(g[:, :3], cur1)
        h = _lfa_h(g[:, 3:], rnf, d1_wnb[...], d1_bnb[...],
                   d1_wmx[...], d1_bmx[...])
        acc = h if acc is None else jnp.maximum(acc, h)
    fea_ref[0] = acc
    coords = []
    for c in range(UPF):
        off = _mm(acc[:, 32 * c:32 * (c + 1)], d1_wp[...]) + d1_bp[...]
        coords.append(cur1 + off)
    coord_ref[0] = jnp.concatenate(coords, axis=1)


def _tail_call(cur, curT, feat, params):
    B = cur.shape[0]
    args = [cur, curT, feat]
    for name in ('enc2a', 'enc2b', 'fin0', 'fin1'):
        p = params[name]
        args += [p['wnb'], p['bnb'].reshape(1, -1),
                 p['wmx'], p['bmx'].reshape(1, -1)]
    args += [params['out_w'], params['out_b'].reshape(1, -1)]
    for name in ('dec0', 'dec1'):
        p = params[name]
        args += [p['wnb'], p['bnb'].reshape(1, -1),
                 p['wmx'], p['bmx'].reshape(1, -1),
                 p['wpred'], p['bpred'].reshape(1, -1)]
    in_specs = [
        pl.BlockSpec((1, 256, 3), lambda b: (b, 0, 0)),
        pl.BlockSpec((1, 3, 256), lambda b: (b, 0, 0)),
        pl.BlockSpec((1, 256, 64), lambda b: (b, 0, 0)),
    ] + [pl.BlockSpec(a.shape, lambda b: (0, 0)) for a in args[3:]]
    return pl.pallas_call(
        _tail_body,
        grid=(B,),
        in_specs=in_specs,
        out_specs=[
            pl.BlockSpec((1, 256, 128), lambda b: (b, 0, 0)),
            pl.BlockSpec((1, 256, 12), lambda b: (b, 0, 0)),
        ],
        out_shape=[
            jax.ShapeDtypeStruct((B, 256, 128), F32),
            jax.ShapeDtypeStruct((B, 256, 12), F32),
        ],
    )(*args)


def _knn_call(cur, curT, bm):
    B, M, _ = cur.shape
    return pl.pallas_call(
        _knn_body,
        grid=(B, M // bm),
        compiler_params=pltpu.CompilerParams(
            dimension_semantics=("parallel", "parallel")),
        in_specs=[
            pl.BlockSpec((1, bm, 3), lambda b, i: (b, i, 0)),
            pl.BlockSpec((1, 3, M), lambda b, i: (b, 0, 0)),
        ],
        out_specs=pl.BlockSpec((1, bm, KNN), lambda b, i: (b, i, 0)),
        out_shape=jax.ShapeDtypeStruct((B, M, KNN), jnp.int32),
    )(cur, curT)


def _lfa2_call(cur_flat, g, p, bm):
    btot = cur_flat.shape[0]
    cout = p['wmx'].shape[-1]
    dcols = g.shape[-1]
    return pl.pallas_call(
        _lfa2_body,
        grid=(btot // bm,),
        compiler_params=pltpu.CompilerParams(
            dimension_semantics=("parallel",)),
        in_specs=[
            pl.BlockSpec((bm, 3), lambda j: (j, 0)),
            pl.BlockSpec((bm * KNN, dcols), lambda j: (j, 0)),
            pl.BlockSpec((10, 16), lambda j: (0, 0)),
            pl.BlockSpec((1, 16), lambda j: (0, 0)),
            pl.BlockSpec(p['wmx'].shape, lambda j: (0, 0)),
            pl.BlockSpec((1, cout), lambda j: (0, 0)),
        ],
        out_specs=pl.BlockSpec((bm, cout), lambda j: (j, 0)),
        out_shape=jax.ShapeDtypeStruct((btot, cout), F32),
    )(cur_flat, g, p['wnb'], p['bnb'].reshape(1, -1),
      p['wmx'], p['bmx'].reshape(1, -1))


def _dec2_call(cur_flat, g, p, bm):
    btot = cur_flat.shape[0]
    cout = p['wmx'].shape[-1]
    dcols = g.shape[-1]
    return pl.pallas_call(
        _dec2_body,
        grid=(btot // bm,),
        compiler_params=pltpu.CompilerParams(
            dimension_semantics=("parallel",)),
        in_specs=[
            pl.BlockSpec((bm, 3), lambda j: (j, 0)),
            pl.BlockSpec((bm * KNN, dcols), lambda j: (j, 0)),
            pl.BlockSpec((10, 16), lambda j: (0, 0)),
            pl.BlockSpec((1, 16), lambda j: (0, 0)),
            pl.BlockSpec(p['wmx'].shape, lambda j: (0, 0)),
            pl.BlockSpec((1, cout), lambda j: (0, 0)),
            pl.BlockSpec(p['wpred'].shape, lambda j: (0, 0)),
            pl.BlockSpec((1, 3), lambda j: (0, 0)),
        ],
        out_specs=[
            pl.BlockSpec((bm, cout), lambda j: (j, 0)),
            pl.BlockSpec((bm, 3 * UPF), lambda j: (j, 0)),
        ],
        out_shape=[
            jax.ShapeDtypeStruct((btot, cout), F32),
            jax.ShapeDtypeStruct((btot, 3 * UPF), F32),
        ],
    )(cur_flat, g, p['wnb'], p['bnb'].reshape(1, -1),
      p['wmx'], p['bmx'].reshape(1, -1),
      p['wpred'], p['bpred'].reshape(1, -1))


def _pad16(x):
    pad = (-x.shape[-1]) % 16
    if pad:
        x = jnp.pad(x, ((0, 0), (0, pad)))
    return x


def _fin_call(cur, curT, feat, p0, p1, ow, ob):
    B, M, _ = cur.shape
    cin = feat.shape[-1]
    comp = ow.shape[-1]
    args = (cur, curT, feat,
            p0['wnb'], p0['bnb'].reshape(1, -1), p0['wmx'], p0['bmx'].reshape(1, -1),
            p1['wnb'], p1['bnb'].reshape(1, -1), p1['wmx'], p1['bmx'].reshape(1, -1),
            ow, ob.reshape(1, -1))
    return pl.pallas_call(
        _fin_body,
        grid=(B,),
        in_specs=[
            pl.BlockSpec((1, M, 3), lambda b: (b, 0, 0)),
            pl.BlockSpec((1, 3, M), lambda b: (b, 0, 0)),
            pl.BlockSpec((1, M, cin), lambda b: (b, 0, 0)),
        ] + [pl.BlockSpec(a.shape, lambda b: (0, 0)) for a in args[3:]],
        out_specs=pl.BlockSpec((1, M, comp), lambda b: (b, 0, 0)),
        out_shape=jax.ShapeDtypeStruct((B, M, comp), F32),
    )(*args)


def kernel(xyz, params):
    B, N, _ = xyz.shape
    cur, feat = xyz, xyz

    for i in range(3):
        M = cur.shape[1]
        curT = jnp.swapaxes(cur, 1, 2)
        idx = _knn_call(cur, curT, min(M, 256))
        idxf = idx.reshape(-1)
        cur_flat = cur.reshape(B * M, 3)
        bm = min(B * M, 256)
        for half in ('a', 'b'):
            table = _pad16(jnp.concatenate(
                [cur_flat, feat.reshape(B * M, -1)], axis=-1))
            g = _sc_gather(table, idxf)
            feat = _lfa2_call(cur_flat, g, params['enc%d%s' % (i, half)], bm)
        feat = feat.reshape(B, M, -1)
        cur, feat = cur[:, ::UPF], feat[:, ::UPF]

    curT = jnp.swapaxes(cur, 1, 2)
    fea = _fin_call(cur, curT, feat, params['fin0'], params['fin1'],
                    params['out_w'], params['out_b'])

    for i in range(3):
        M = cur.shape[1]
        curT = jnp.swapaxes(cur, 1, 2)
        idx = _knn_call(cur, curT, min(M, 256))
        idxf = idx.reshape(-1)
        cur_flat = cur.reshape(B * M, 3)
        table = _pad16(jnp.concatenate(
            [cur_flat, fea.reshape(B * M, -1)], axis=-1))
        g = _sc_gather(table, idxf)
        f, coord12 = _dec2_call(cur_flat, g, params['dec%d' % i],
                                min(B * M, 256))
        fea = f.reshape(B, M * UPF, 32)
        cur = coord12.reshape(B, M * UPF, 3)
    return fea
